# baseline (speedup 1.0000x reference)
import functools

import jax
import jax.numpy as jnp
from jax import lax
from jax.experimental import pallas as pl
from jax.experimental.pallas import tpu as pltpu
from jax.experimental.pallas import tpu_sc as plsc

_N = 4194304
_NC = 2
_NS = 16
_NW = _NC * _NS
_CH = 16384
_NCH = 3
_PW = _NCH * _CH
_NSC = _NW * _PW
_BLK = 1024
_SPB = _BLK // 16
_BPC = _CH // _BLK
_NBLK = _PW // _BLK

_BR = 512
_R0 = _NSC // 128
_GT = (_N // 128 - _R0) // _BR

_NEG = -3.4e38
_BIGI = 2**30


def _lane_max(vec):
    idx = lax.iota(jnp.int32, 16)
    dnums = lax.GatherDimensionNumbers(
        offset_dims=(), collapsed_slice_dims=(0,), start_index_map=(0,))
    for sh in (8, 4, 2, 1):
        perm = jnp.bitwise_xor(idx, sh)
        shuf = lax.gather(vec, perm[:, None], dnums, slice_sizes=(1,),
                          unique_indices=True, indices_are_sorted=False,
                          mode=lax.GatherScatterMode.PROMISE_IN_BOUNDS)
        vec = jnp.maximum(vec, shuf)
    return vec[0]


def _scan_body(in_hbm, tg_hbm, v_out, i_out, a_out, b_out,
               in_a, in_b, tg_a, tg_b, bmax,
               vscr, iscr, ascr, bscr, sem_a, sem_b):
    cid = lax.axis_index("c")
    sid = lax.axis_index("s")
    wid = sid * _NC + cid
    base = wid * _PW

    in_bufs = (in_a, in_b)
    tg_bufs = (tg_a, tg_b)
    sems = (sem_a, sem_b)

    def fire(c):
        par = c % 2
        cpa = pltpu.make_async_copy(
            in_hbm.at[pl.ds(base + c * _CH, _CH)], in_bufs[par], sems[par])
        cpb = pltpu.make_async_copy(
            tg_hbm.at[pl.ds(base + c * _CH, _CH)], tg_bufs[par], sems[par])
        cpa.start()
        cpb.start()
        return cpa, cpb

    pend = fire(0)
    vglob = jnp.zeros((16,), jnp.float32)
    for c in range(_NCH):
        nxt = fire(c + 1) if c + 1 < _NCH else None
        pend[0].wait()
        pend[1].wait()
        pend = nxt
        ibuf = in_bufs[c % 2]
        tbuf = tg_bufs[c % 2]

        @plsc.parallel_loop(0, _BPC, carry=vglob)
        def blk_body(bi, vg, ibuf=ibuf, tbuf=tbuf, c=c):
            accs = [jnp.zeros((16,), jnp.float32) for _ in range(4)]
            for s in range(_SPB):
                off = bi * _BLK + s * 16
                a = ibuf[pl.ds(off, 16)]
                t = tbuf[pl.ds(off, 16)]
                d = a - t
                accs[s % 4] = jnp.maximum(accs[s % 4], d * d)
            vmax = jnp.maximum(jnp.maximum(accs[0], accs[1]),
                               jnp.maximum(accs[2], accs[3]))
            bmax[pl.ds((c * _BPC + bi) * 16, 16)] = vmax
            return jnp.maximum(vg, vmax)

        vglob = blk_body

    m_val = _lane_max(vglob)

    def red_body(b, bf):
        vec = bmax[pl.ds(b * 16, 16)]
        hitv = (vec == m_val) & (bf == _NBLK)
        return jnp.where(hitv, b, bf)

    bfirst = lax.fori_loop(
        0, _NBLK, red_body, jnp.full((16,), _NBLK, jnp.int32))
    b_star = (-_lane_max(-bfirst.astype(jnp.float32))).astype(jnp.int32)
    b_star = jnp.minimum(b_star, _NBLK - 1)

    gbase = base + b_star * _BLK
    cpa = pltpu.make_async_copy(
        in_hbm.at[pl.ds(gbase, _BLK)], in_a.at[pl.ds(0, _BLK)], sem_a)
    cpb = pltpu.make_async_copy(
        tg_hbm.at[pl.ds(gbase, _BLK)], tg_a.at[pl.ds(0, _BLK)], sem_a)
    cpa.start()
    cpb.start()
    cpa.wait()
    cpb.wait()

    lane = lax.iota(jnp.int32, 16)

    def rs_body(si, carry):
        bidx, b_a, b_b = carry
        a = in_a[pl.ds(si * 16, 16)]
        t = tg_a[pl.ds(si * 16, 16)]
        d = a - t
        d2 = d * d
        idxv = gbase + si * 16 + lane
        hit = (d2 == m_val) & (idxv < bidx)
        return (jnp.where(hit, idxv, bidx),
                jnp.where(hit, a, b_a),
                jnp.where(hit, t, b_b))

    bidx, b_a, b_b = lax.fori_loop(
        0, _SPB, rs_body,
        (jnp.full((16,), _BIGI, jnp.int32),
         jnp.zeros((16,), jnp.float32),
         jnp.zeros((16,), jnp.float32)))

    found = bidx < _BIGI
    vscr[...] = jnp.where(found, m_val, _NEG)
    iscr[...] = bidx
    ascr[...] = b_a
    bscr[...] = b_b
    pltpu.sync_copy(vscr, v_out.at[wid])
    pltpu.sync_copy(iscr, i_out.at[wid])
    pltpu.sync_copy(ascr, a_out.at[wid])
    pltpu.sync_copy(bscr, b_out.at[wid])


_phase1 = pl.kernel(
    _scan_body,
    out_type=[
        jax.ShapeDtypeStruct((_NW, 16), jnp.float32),
        jax.ShapeDtypeStruct((_NW, 16), jnp.int32),
        jax.ShapeDtypeStruct((_NW, 16), jnp.float32),
        jax.ShapeDtypeStruct((_NW, 16), jnp.float32),
    ],
    mesh=plsc.VectorSubcoreMesh(
        core_axis_name="c", subcore_axis_name="s",
        num_cores=_NC, num_subcores=_NS),
    scratch_types=[
        pltpu.VMEM((_CH,), jnp.float32),
        pltpu.VMEM((_CH,), jnp.float32),
        pltpu.VMEM((_CH,), jnp.float32),
        pltpu.VMEM((_CH,), jnp.float32),
        pltpu.VMEM((_NBLK * 16,), jnp.float32),
        pltpu.VMEM((16,), jnp.float32),
        pltpu.VMEM((16,), jnp.int32),
        pltpu.VMEM((16,), jnp.float32),
        pltpu.VMEM((16,), jnp.float32),
        pltpu.SemaphoreType.DMA,
        pltpu.SemaphoreType.DMA,
    ],
)


def _tc_body(in_ref, tg_ref, vo, io, ao, bo, vmax_s, sidx_s, va_s, vb_s):
    i = pl.program_id(0)

    @pl.when(i == 0)
    def _():
        vmax_s[...] = jnp.full((_BR, 128), -1.0, jnp.float32)
        sidx_s[...] = jnp.zeros((_BR, 128), jnp.int32)
        va_s[...] = jnp.zeros((_BR, 128), jnp.float32)
        vb_s[...] = jnp.zeros((_BR, 128), jnp.float32)

    a = in_ref[...]
    t = tg_ref[...]
    d = a - t
    d2 = d * d
    m = d2 > vmax_s[...]
    vmax_s[...] = jnp.where(m, d2, vmax_s[...])
    sidx_s[...] = jnp.where(m, i, sidx_s[...])
    va_s[...] = jnp.where(m, a, va_s[...])
    vb_s[...] = jnp.where(m, t, vb_s[...])

    @pl.when(i == _GT - 1)
    def _():
        vm = vmax_s[...]
        m_g = jnp.max(vm)
        rows = lax.broadcasted_iota(jnp.int32, (_BR, 128), 0)
        cols = lax.broadcasted_iota(jnp.int32, (_BR, 128), 1)
        idxf = (_R0 + sidx_s[...] * _BR + rows) * 128 + cols
        eq = vm == m_g
        g_i = jnp.min(jnp.where(eq, idxf, _BIGI))
        sel = eq & (idxf == g_i)
        vo[0] = m_g
        io[0] = g_i
        ao[0] = jnp.max(jnp.where(sel, va_s[...], _NEG))
        bo[0] = jnp.max(jnp.where(sel, vb_s[...], _NEG))


_tc_scan = pl.pallas_call(
    _tc_body,
    grid=(_GT,),
    in_specs=[
        pl.BlockSpec((_BR, 128), lambda i: (_R0 // _BR + i, 0)),
        pl.BlockSpec((_BR, 128), lambda i: (_R0 // _BR + i, 0)),
    ],
    out_shape=[
        jax.ShapeDtypeStruct((1,), jnp.float32),
        jax.ShapeDtypeStruct((1,), jnp.int32),
        jax.ShapeDtypeStruct((1,), jnp.float32),
        jax.ShapeDtypeStruct((1,), jnp.float32),
    ],
    out_specs=[
        pl.BlockSpec(memory_space=pltpu.SMEM),
        pl.BlockSpec(memory_space=pltpu.SMEM),
        pl.BlockSpec(memory_space=pltpu.SMEM),
        pl.BlockSpec(memory_space=pltpu.SMEM),
    ],
    scratch_shapes=[
        pltpu.VMEM((_BR, 128), jnp.float32),
        pltpu.VMEM((_BR, 128), jnp.int32),
        pltpu.VMEM((_BR, 128), jnp.float32),
        pltpu.VMEM((_BR, 128), jnp.float32),
    ],
)


def _merge_body(v_ref, i_ref, a_ref, b_ref, vt_ref, at_ref, bt_ref,
                md_ref, p_ref, ac_ref):
    v = v_ref[...]
    idx = i_ref[...]
    a = a_ref[...]
    b = b_ref[...]
    m_g = jnp.max(v)
    maskv = v == m_g
    g_i = jnp.min(jnp.where(maskv, idx, _BIGI))
    sel = maskv & (idx == g_i)
    a_sc = jnp.max(jnp.where(sel, a, _NEG))
    b_sc = jnp.max(jnp.where(sel, b, _NEG))
    take_tc = vt_ref[0] > m_g
    m_w = jnp.where(take_tc, vt_ref[0], m_g)
    md_ref[0] = jnp.sqrt(m_w)
    p_ref[0] = jnp.where(take_tc, at_ref[0], a_sc)
    ac_ref[0] = jnp.where(take_tc, bt_ref[0], b_sc)


_phase2 = pl.pallas_call(
    _merge_body,
    in_specs=[
        pl.BlockSpec(memory_space=pltpu.MemorySpace.VMEM),
        pl.BlockSpec(memory_space=pltpu.MemorySpace.VMEM),
        pl.BlockSpec(memory_space=pltpu.MemorySpace.VMEM),
        pl.BlockSpec(memory_space=pltpu.MemorySpace.VMEM),
        pl.BlockSpec(memory_space=pltpu.SMEM),
        pl.BlockSpec(memory_space=pltpu.SMEM),
        pl.BlockSpec(memory_space=pltpu.SMEM),
    ],
    out_shape=[
        jax.ShapeDtypeStruct((1,), jnp.float32),
        jax.ShapeDtypeStruct((1,), jnp.float32),
        jax.ShapeDtypeStruct((1,), jnp.float32),
    ],
    out_specs=[
        pl.BlockSpec(memory_space=pltpu.SMEM),
        pl.BlockSpec(memory_space=pltpu.SMEM),
        pl.BlockSpec(memory_space=pltpu.SMEM),
    ],
)


def kernel(inputs, target):
    v, idx, a, b = _phase1(inputs, target)
    vt, _, at, bt = _tc_scan(inputs.reshape(-1, 128), target.reshape(-1, 128))
    md, p, ac = _phase2(v, idx, a, b, vt, at, bt)
    return (md[0], p[0], ac[0])


# speedup vs baseline: 1.0010x; 1.0010x over previous
import functools

import jax
import jax.numpy as jnp
from jax import lax
from jax.experimental import pallas as pl
from jax.experimental.pallas import tpu as pltpu
from jax.experimental.pallas import tpu_sc as plsc

_N = 4194304
_NC = 2
_NS = 16
_NW = _NC * _NS
_CH = 16384
_NCH = 3
_PW = _NCH * _CH
_NSC = _NW * _PW
_BLK = 1024
_SPB = _BLK // 16
_BPC = _CH // _BLK
_NBLK = _PW // _BLK

_BR = 512
_R0 = _NSC // 128
_GT = (_N // 128 - _R0) // _BR

_NEG = -3.4e38
_BIGI = 2**30


def _lane_max(vec):
    idx = lax.iota(jnp.int32, 16)
    dnums = lax.GatherDimensionNumbers(
        offset_dims=(), collapsed_slice_dims=(0,), start_index_map=(0,))
    for sh in (8, 4, 2, 1):
        perm = jnp.bitwise_xor(idx, sh)
        shuf = lax.gather(vec, perm[:, None], dnums, slice_sizes=(1,),
                          unique_indices=True, indices_are_sorted=False,
                          mode=lax.GatherScatterMode.PROMISE_IN_BOUNDS)
        vec = jnp.maximum(vec, shuf)
    return vec[0]


def _scan_body(in_hbm, tg_hbm, v_out, i_out, a_out, b_out,
               in_a, in_b, tg_a, tg_b, bmax,
               vscr, iscr, ascr, bscr, sem_a, sem_b):
    cid = lax.axis_index("c")
    sid = lax.axis_index("s")
    wid = sid * _NC + cid
    base = wid * _PW

    in_bufs = (in_a, in_b)
    tg_bufs = (tg_a, tg_b)
    sems = (sem_a, sem_b)

    def fire(c):
        par = c % 2
        cpa = pltpu.make_async_copy(
            in_hbm.at[pl.ds(base + c * _CH, _CH)], in_bufs[par], sems[par])
        cpb = pltpu.make_async_copy(
            tg_hbm.at[pl.ds(base + c * _CH, _CH)], tg_bufs[par], sems[par])
        cpa.start()
        cpb.start()
        return cpa, cpb

    pend = fire(0)
    vglob = jnp.zeros((16,), jnp.float32)
    for c in range(_NCH):
        nxt = fire(c + 1) if c + 1 < _NCH else None
        pend[0].wait()
        pend[1].wait()
        pend = nxt
        ibuf = in_bufs[c % 2]
        tbuf = tg_bufs[c % 2]

        @plsc.parallel_loop(0, _BPC, carry=vglob)
        def blk_body(bi, vg, ibuf=ibuf, tbuf=tbuf, c=c):
            accs = [jnp.zeros((16,), jnp.float32) for _ in range(4)]
            for s in range(_SPB):
                off = bi * _BLK + s * 16
                a = ibuf[pl.ds(off, 16)]
                t = tbuf[pl.ds(off, 16)]
                d = a - t
                accs[s % 4] = jnp.maximum(accs[s % 4], d * d)
            vmax = jnp.maximum(jnp.maximum(accs[0], accs[1]),
                               jnp.maximum(accs[2], accs[3]))
            bmax[pl.ds((c * _BPC + bi) * 16, 16)] = vmax
            return jnp.maximum(vg, vmax)

        vglob = blk_body

    m_val = _lane_max(vglob)

    def red_body(b, bf):
        vec = bmax[pl.ds(b * 16, 16)]
        hitv = (vec == m_val) & (bf == _NBLK)
        return jnp.where(hitv, b, bf)

    bfirst = lax.fori_loop(
        0, _NBLK, red_body, jnp.full((16,), _NBLK, jnp.int32))
    b_star = (-_lane_max(-bfirst.astype(jnp.float32))).astype(jnp.int32)
    b_star = jnp.minimum(b_star, _NBLK - 1)

    gbase = base + b_star * _BLK
    cpa = pltpu.make_async_copy(
        in_hbm.at[pl.ds(gbase, _BLK)], in_a.at[pl.ds(0, _BLK)], sem_a)
    cpb = pltpu.make_async_copy(
        tg_hbm.at[pl.ds(gbase, _BLK)], tg_a.at[pl.ds(0, _BLK)], sem_a)
    cpa.start()
    cpb.start()
    cpa.wait()
    cpb.wait()

    lane = lax.iota(jnp.int32, 16)

    def rs_body(si, carry):
        bidx, b_a, b_b = carry
        a = in_a[pl.ds(si * 16, 16)]
        t = tg_a[pl.ds(si * 16, 16)]
        d = a - t
        d2 = d * d
        idxv = gbase + si * 16 + lane
        hit = (d2 == m_val) & (idxv < bidx)
        return (jnp.where(hit, idxv, bidx),
                jnp.where(hit, a, b_a),
                jnp.where(hit, t, b_b))

    bidx, b_a, b_b = lax.fori_loop(
        0, _SPB, rs_body,
        (jnp.full((16,), _BIGI, jnp.int32),
         jnp.zeros((16,), jnp.float32),
         jnp.zeros((16,), jnp.float32)))

    found = bidx < _BIGI
    vscr[...] = jnp.where(found, m_val, _NEG)
    iscr[...] = bidx
    ascr[...] = b_a
    bscr[...] = b_b
    pltpu.sync_copy(vscr, v_out.at[wid])
    pltpu.sync_copy(iscr, i_out.at[wid])
    pltpu.sync_copy(ascr, a_out.at[wid])
    pltpu.sync_copy(bscr, b_out.at[wid])


_phase1 = pl.kernel(
    _scan_body,
    out_type=[
        jax.ShapeDtypeStruct((_NW, 16), jnp.float32),
        jax.ShapeDtypeStruct((_NW, 16), jnp.int32),
        jax.ShapeDtypeStruct((_NW, 16), jnp.float32),
        jax.ShapeDtypeStruct((_NW, 16), jnp.float32),
    ],
    mesh=plsc.VectorSubcoreMesh(
        core_axis_name="c", subcore_axis_name="s",
        num_cores=_NC, num_subcores=_NS),
    scratch_types=[
        pltpu.VMEM((_CH,), jnp.float32),
        pltpu.VMEM((_CH,), jnp.float32),
        pltpu.VMEM((_CH,), jnp.float32),
        pltpu.VMEM((_CH,), jnp.float32),
        pltpu.VMEM((_NBLK * 16,), jnp.float32),
        pltpu.VMEM((16,), jnp.float32),
        pltpu.VMEM((16,), jnp.int32),
        pltpu.VMEM((16,), jnp.float32),
        pltpu.VMEM((16,), jnp.float32),
        pltpu.SemaphoreType.DMA,
        pltpu.SemaphoreType.DMA,
    ],
)


def _tc_body(in_ref, tg_ref, bm_ref, mo_ref, so_ref):
    i = pl.program_id(0)
    d = in_ref[...] - tg_ref[...]
    bm_ref[i] = jnp.max(d * d)

    @pl.when(i == _GT - 1)
    def _():
        def red(s, carry):
            m_cur, s_cur = carry
            m_s = bm_ref[s]
            take = m_s > m_cur
            return (jnp.where(take, m_s, m_cur), jnp.where(take, s, s_cur))

        m_tc, s_tc = lax.fori_loop(
            0, _GT, red, (jnp.float32(-1.0), jnp.int32(0)))
        mo_ref[0] = m_tc
        so_ref[0] = s_tc


_tc_scan = pl.pallas_call(
    _tc_body,
    grid=(_GT,),
    in_specs=[
        pl.BlockSpec((_BR, 128), lambda i: (_R0 // _BR + i, 0)),
        pl.BlockSpec((_BR, 128), lambda i: (_R0 // _BR + i, 0)),
    ],
    out_shape=[
        jax.ShapeDtypeStruct((_GT,), jnp.float32),
        jax.ShapeDtypeStruct((1,), jnp.float32),
        jax.ShapeDtypeStruct((1,), jnp.int32),
    ],
    out_specs=[
        pl.BlockSpec(memory_space=pltpu.SMEM),
        pl.BlockSpec(memory_space=pltpu.SMEM),
        pl.BlockSpec(memory_space=pltpu.SMEM),
    ],
)


def _merge_body(s_ref, in_ref, tg_ref, mt_ref, v_ref, i_ref, a_ref, b_ref,
                md_ref, p_ref, ac_ref):
    v = v_ref[...]
    idx = i_ref[...]
    m_sc = jnp.max(v)
    maskv = v == m_sc
    g_sc = jnp.min(jnp.where(maskv, idx, _BIGI))
    sel = maskv & (idx == g_sc)
    a_sc = jnp.max(jnp.where(sel, a_ref[...], _NEG))
    b_sc = jnp.max(jnp.where(sel, b_ref[...], _NEG))
    m_tc = mt_ref[0]
    a_blk = in_ref[...]
    t_blk = tg_ref[...]
    d = a_blk - t_blk
    d2 = d * d
    rows = lax.broadcasted_iota(jnp.int32, (_BR, 128), 0)
    cols = lax.broadcasted_iota(jnp.int32, (_BR, 128), 1)
    idxf = (_R0 + s_ref[0] * _BR + rows) * 128 + cols
    eq = d2 == m_tc
    g_tc = jnp.min(jnp.where(eq, idxf, _BIGI))
    selt = eq & (idxf == g_tc)
    a_tc = jnp.max(jnp.where(selt, a_blk, _NEG))
    b_tc = jnp.max(jnp.where(selt, t_blk, _NEG))
    take_tc = m_tc > m_sc
    md_ref[0] = jnp.sqrt(jnp.where(take_tc, m_tc, m_sc))
    p_ref[0] = jnp.where(take_tc, a_tc, a_sc)
    ac_ref[0] = jnp.where(take_tc, b_tc, b_sc)


_phase2 = pl.pallas_call(
    _merge_body,
    grid_spec=pltpu.PrefetchScalarGridSpec(
        num_scalar_prefetch=1,
        grid=(1,),
        in_specs=[
            pl.BlockSpec((_BR, 128), lambda i, s: (_R0 // _BR + s[0], 0)),
            pl.BlockSpec((_BR, 128), lambda i, s: (_R0 // _BR + s[0], 0)),
            pl.BlockSpec(memory_space=pltpu.SMEM),
            pl.BlockSpec(memory_space=pltpu.MemorySpace.VMEM),
            pl.BlockSpec(memory_space=pltpu.MemorySpace.VMEM),
            pl.BlockSpec(memory_space=pltpu.MemorySpace.VMEM),
            pl.BlockSpec(memory_space=pltpu.MemorySpace.VMEM),
        ],
        out_specs=[
            pl.BlockSpec(memory_space=pltpu.SMEM),
            pl.BlockSpec(memory_space=pltpu.SMEM),
            pl.BlockSpec(memory_space=pltpu.SMEM),
        ],
    ),
    out_shape=[
        jax.ShapeDtypeStruct((1,), jnp.float32),
        jax.ShapeDtypeStruct((1,), jnp.float32),
        jax.ShapeDtypeStruct((1,), jnp.float32),
    ],
)


def kernel(inputs, target):
    in2 = inputs.reshape(-1, 128)
    tg2 = target.reshape(-1, 128)
    v, idx, a, b = _phase1(inputs, target)
    _, mt, st = _tc_scan(in2, tg2)
    md, p, ac = _phase2(st, in2, tg2, mt, v, idx, a, b)
    return (md[0], p[0], ac[0])


# speedup vs baseline: 1.0076x; 1.0066x over previous
import functools

import jax
import jax.numpy as jnp
from jax import lax
from jax.experimental import pallas as pl
from jax.experimental.pallas import tpu as pltpu
from jax.experimental.pallas import tpu_sc as plsc

_N = 4194304
_NC = 2
_NS = 16
_NW = _NC * _NS
_CH = 16384
_NCH = 3
_PW = _NCH * _CH
_NSC = _NW * _PW
_BLK = 1024
_SPB = _BLK // 16
_BPC = _CH // _BLK
_NBLK = _PW // _BLK

_BR = 512
_CB = _BR * 128
_B0 = _NSC // _CB
_GT = (_N - _NSC) // _CB

_NEG = -3.4e38
_BIGI = 2**30


def _lane_max(vec):
    idx = lax.iota(jnp.int32, 16)
    dnums = lax.GatherDimensionNumbers(
        offset_dims=(), collapsed_slice_dims=(0,), start_index_map=(0,))
    for sh in (8, 4, 2, 1):
        perm = jnp.bitwise_xor(idx, sh)
        shuf = lax.gather(vec, perm[:, None], dnums, slice_sizes=(1,),
                          unique_indices=True, indices_are_sorted=False,
                          mode=lax.GatherScatterMode.PROMISE_IN_BOUNDS)
        vec = jnp.maximum(vec, shuf)
    return vec[0]


def _scan_body(in_hbm, tg_hbm, v_out, i_out, a_out, b_out,
               in_a, in_b, tg_a, tg_b, bmax,
               vscr, iscr, ascr, bscr, sem_a, sem_b):
    cid = lax.axis_index("c")
    sid = lax.axis_index("s")
    wid = sid * _NC + cid
    base = wid * _PW

    in_bufs = (in_a, in_b)
    tg_bufs = (tg_a, tg_b)
    sems = (sem_a, sem_b)

    def fire(c):
        par = c % 2
        cpa = pltpu.make_async_copy(
            in_hbm.at[pl.ds(base + c * _CH, _CH)], in_bufs[par], sems[par])
        cpb = pltpu.make_async_copy(
            tg_hbm.at[pl.ds(base + c * _CH, _CH)], tg_bufs[par], sems[par])
        cpa.start()
        cpb.start()
        return cpa, cpb

    pend = fire(0)
    vglob = jnp.zeros((16,), jnp.float32)
    for c in range(_NCH):
        nxt = fire(c + 1) if c + 1 < _NCH else None
        pend[0].wait()
        pend[1].wait()
        pend = nxt
        ibuf = in_bufs[c % 2]
        tbuf = tg_bufs[c % 2]

        @plsc.parallel_loop(0, _BPC, carry=vglob)
        def blk_body(bi, vg, ibuf=ibuf, tbuf=tbuf, c=c):
            accs = [jnp.zeros((16,), jnp.float32) for _ in range(4)]
            for s in range(_SPB):
                off = bi * _BLK + s * 16
                a = ibuf[pl.ds(off, 16)]
                t = tbuf[pl.ds(off, 16)]
                d = a - t
                accs[s % 4] = jnp.maximum(accs[s % 4], d * d)
            vmax = jnp.maximum(jnp.maximum(accs[0], accs[1]),
                               jnp.maximum(accs[2], accs[3]))
            bmax[pl.ds((c * _BPC + bi) * 16, 16)] = vmax
            return jnp.maximum(vg, vmax)

        vglob = blk_body

    m_val = _lane_max(vglob)

    def red_body(b, bf):
        vec = bmax[pl.ds(b * 16, 16)]
        hitv = (vec == m_val) & (bf == _NBLK)
        return jnp.where(hitv, b, bf)

    bfirst = lax.fori_loop(
        0, _NBLK, red_body, jnp.full((16,), _NBLK, jnp.int32))
    b_star = (-_lane_max(-bfirst.astype(jnp.float32))).astype(jnp.int32)
    b_star = jnp.minimum(b_star, _NBLK - 1)

    gbase = base + b_star * _BLK
    cpa = pltpu.make_async_copy(
        in_hbm.at[pl.ds(gbase, _BLK)], in_a.at[pl.ds(0, _BLK)], sem_a)
    cpb = pltpu.make_async_copy(
        tg_hbm.at[pl.ds(gbase, _BLK)], tg_a.at[pl.ds(0, _BLK)], sem_a)
    cpa.start()
    cpb.start()
    cpa.wait()
    cpb.wait()

    lane = lax.iota(jnp.int32, 16)

    def rs_body(si, carry):
        bidx, b_a, b_b = carry
        a = in_a[pl.ds(si * 16, 16)]
        t = tg_a[pl.ds(si * 16, 16)]
        d = a - t
        d2 = d * d
        idxv = gbase + si * 16 + lane
        hit = (d2 == m_val) & (idxv < bidx)
        return (jnp.where(hit, idxv, bidx),
                jnp.where(hit, a, b_a),
                jnp.where(hit, t, b_b))

    bidx, b_a, b_b = lax.fori_loop(
        0, _SPB, rs_body,
        (jnp.full((16,), _BIGI, jnp.int32),
         jnp.zeros((16,), jnp.float32),
         jnp.zeros((16,), jnp.float32)))

    found = bidx < _BIGI
    vscr[...] = jnp.where(found, m_val, _NEG)
    iscr[...] = bidx
    ascr[...] = b_a
    bscr[...] = b_b
    pltpu.sync_copy(vscr, v_out.at[wid])
    pltpu.sync_copy(iscr, i_out.at[wid])
    pltpu.sync_copy(ascr, a_out.at[wid])
    pltpu.sync_copy(bscr, b_out.at[wid])


_phase1 = pl.kernel(
    _scan_body,
    out_type=[
        jax.ShapeDtypeStruct((_NW, 16), jnp.float32),
        jax.ShapeDtypeStruct((_NW, 16), jnp.int32),
        jax.ShapeDtypeStruct((_NW, 16), jnp.float32),
        jax.ShapeDtypeStruct((_NW, 16), jnp.float32),
    ],
    mesh=plsc.VectorSubcoreMesh(
        core_axis_name="c", subcore_axis_name="s",
        num_cores=_NC, num_subcores=_NS),
    scratch_types=[
        pltpu.VMEM((_CH,), jnp.float32),
        pltpu.VMEM((_CH,), jnp.float32),
        pltpu.VMEM((_CH,), jnp.float32),
        pltpu.VMEM((_CH,), jnp.float32),
        pltpu.VMEM((_NBLK * 16,), jnp.float32),
        pltpu.VMEM((16,), jnp.float32),
        pltpu.VMEM((16,), jnp.int32),
        pltpu.VMEM((16,), jnp.float32),
        pltpu.VMEM((16,), jnp.float32),
        pltpu.SemaphoreType.DMA,
        pltpu.SemaphoreType.DMA,
    ],
)


def _tc_body(in_ref, tg_ref, bm_ref, mo_ref, so_ref):
    i = pl.program_id(0)
    d = in_ref[...].reshape(_BR, 128) - tg_ref[...].reshape(_BR, 128)
    bm_ref[i] = jnp.max(d * d)

    @pl.when(i == _GT - 1)
    def _():
        def red(s, carry):
            m_cur, s_cur = carry
            m_s = bm_ref[s]
            take = m_s > m_cur
            return (jnp.where(take, m_s, m_cur), jnp.where(take, s, s_cur))

        m_tc, s_tc = lax.fori_loop(
            0, _GT, red, (jnp.float32(-1.0), jnp.int32(0)))
        mo_ref[0] = m_tc
        so_ref[0] = s_tc


_tc_scan = pl.pallas_call(
    _tc_body,
    grid=(_GT,),
    in_specs=[
        pl.BlockSpec((_CB,), lambda i: (_B0 + i,)),
        pl.BlockSpec((_CB,), lambda i: (_B0 + i,)),
    ],
    out_shape=[
        jax.ShapeDtypeStruct((_GT,), jnp.float32),
        jax.ShapeDtypeStruct((1,), jnp.float32),
        jax.ShapeDtypeStruct((1,), jnp.int32),
    ],
    out_specs=[
        pl.BlockSpec(memory_space=pltpu.SMEM),
        pl.BlockSpec(memory_space=pltpu.SMEM),
        pl.BlockSpec(memory_space=pltpu.SMEM),
    ],
)


def _merge_body(s_ref, in_ref, tg_ref, mt_ref, v_ref, i_ref, a_ref, b_ref,
                md_ref, p_ref, ac_ref):
    v = v_ref[...]
    idx = i_ref[...]
    m_sc = jnp.max(v)
    maskv = v == m_sc
    g_sc = jnp.min(jnp.where(maskv, idx, _BIGI))
    sel = maskv & (idx == g_sc)
    a_sc = jnp.max(jnp.where(sel, a_ref[...], _NEG))
    b_sc = jnp.max(jnp.where(sel, b_ref[...], _NEG))
    m_tc = mt_ref[0]
    a_blk = in_ref[...].reshape(_BR, 128)
    t_blk = tg_ref[...].reshape(_BR, 128)
    d = a_blk - t_blk
    d2 = d * d
    rows = lax.broadcasted_iota(jnp.int32, (_BR, 128), 0)
    cols = lax.broadcasted_iota(jnp.int32, (_BR, 128), 1)
    idxf = _NSC + s_ref[0] * _CB + rows * 128 + cols
    eq = d2 == m_tc
    g_tc = jnp.min(jnp.where(eq, idxf, _BIGI))
    selt = eq & (idxf == g_tc)
    a_tc = jnp.max(jnp.where(selt, a_blk, _NEG))
    b_tc = jnp.max(jnp.where(selt, t_blk, _NEG))
    take_tc = m_tc > m_sc
    md_ref[0] = jnp.sqrt(jnp.where(take_tc, m_tc, m_sc))
    p_ref[0] = jnp.where(take_tc, a_tc, a_sc)
    ac_ref[0] = jnp.where(take_tc, b_tc, b_sc)


_phase2 = pl.pallas_call(
    _merge_body,
    grid_spec=pltpu.PrefetchScalarGridSpec(
        num_scalar_prefetch=1,
        grid=(1,),
        in_specs=[
            pl.BlockSpec((_CB,), lambda i, s: (_B0 + s[0],)),
            pl.BlockSpec((_CB,), lambda i, s: (_B0 + s[0],)),
            pl.BlockSpec(memory_space=pltpu.SMEM),
            pl.BlockSpec(memory_space=pltpu.MemorySpace.VMEM),
            pl.BlockSpec(memory_space=pltpu.MemorySpace.VMEM),
            pl.BlockSpec(memory_space=pltpu.MemorySpace.VMEM),
            pl.BlockSpec(memory_space=pltpu.MemorySpace.VMEM),
        ],
        out_specs=[
            pl.BlockSpec(memory_space=pltpu.SMEM),
            pl.BlockSpec(memory_space=pltpu.SMEM),
            pl.BlockSpec(memory_space=pltpu.SMEM),
        ],
    ),
    out_shape=[
        jax.ShapeDtypeStruct((1,), jnp.float32),
        jax.ShapeDtypeStruct((1,), jnp.float32),
        jax.ShapeDtypeStruct((1,), jnp.float32),
    ],
)


def kernel(inputs, target):
    v, idx, a, b = _phase1(inputs, target)
    _, mt, st = _tc_scan(inputs, target)
    md, p, ac = _phase2(st, inputs, target, mt, v, idx, a, b)
    return (md[0], p[0], ac[0])


# speedup vs baseline: 1.2985x; 1.2887x over previous
import functools

import jax
import jax.numpy as jnp
from jax import lax
from jax.experimental import pallas as pl
from jax.experimental.pallas import tpu as pltpu
from jax.experimental.pallas import tpu_sc as plsc

_N = 4194304
_NC = 2
_NS = 16
_NW = _NC * _NS
_CH = 16384
_NCH = 3
_PW = _NCH * _CH
_NSC = _NW * _PW
_BLK = 1024
_SPB = _BLK // 16
_BPC = _CH // _BLK
_NBLK = _PW // _BLK

_CB = 524288
_RB = 65536
_SB = _CB // _RB
_BR = _RB // 128
_B0 = _NSC // _CB
_GT = (_N - _NSC) // _CB
_NBM = _GT * _SB

_NEG = -3.4e38
_BIGI = 2**30


def _lane_max(vec):
    idx = lax.iota(jnp.int32, 16)
    dnums = lax.GatherDimensionNumbers(
        offset_dims=(), collapsed_slice_dims=(0,), start_index_map=(0,))
    for sh in (8, 4, 2, 1):
        perm = jnp.bitwise_xor(idx, sh)
        shuf = lax.gather(vec, perm[:, None], dnums, slice_sizes=(1,),
                          unique_indices=True, indices_are_sorted=False,
                          mode=lax.GatherScatterMode.PROMISE_IN_BOUNDS)
        vec = jnp.maximum(vec, shuf)
    return vec[0]


def _scan_body(in_hbm, tg_hbm, v_out, i_out, a_out, b_out,
               in_a, in_b, tg_a, tg_b, bmax,
               vscr, iscr, ascr, bscr, sem_a, sem_b):
    cid = lax.axis_index("c")
    sid = lax.axis_index("s")
    wid = sid * _NC + cid
    base = wid * _PW

    in_bufs = (in_a, in_b)
    tg_bufs = (tg_a, tg_b)
    sems = (sem_a, sem_b)

    def fire(c):
        par = c % 2
        cpa = pltpu.make_async_copy(
            in_hbm.at[pl.ds(base + c * _CH, _CH)], in_bufs[par], sems[par])
        cpb = pltpu.make_async_copy(
            tg_hbm.at[pl.ds(base + c * _CH, _CH)], tg_bufs[par], sems[par])
        cpa.start()
        cpb.start()
        return cpa, cpb

    pend = fire(0)
    vglob = jnp.zeros((16,), jnp.float32)
    for c in range(_NCH):
        nxt = fire(c + 1) if c + 1 < _NCH else None
        pend[0].wait()
        pend[1].wait()
        pend = nxt
        ibuf = in_bufs[c % 2]
        tbuf = tg_bufs[c % 2]

        @plsc.parallel_loop(0, _BPC, carry=vglob)
        def blk_body(bi, vg, ibuf=ibuf, tbuf=tbuf, c=c):
            accs = [jnp.zeros((16,), jnp.float32) for _ in range(4)]
            for s in range(_SPB):
                off = bi * _BLK + s * 16
                a = ibuf[pl.ds(off, 16)]
                t = tbuf[pl.ds(off, 16)]
                d = a - t
                accs[s % 4] = jnp.maximum(accs[s % 4], d * d)
            vmax = jnp.maximum(jnp.maximum(accs[0], accs[1]),
                               jnp.maximum(accs[2], accs[3]))
            bmax[pl.ds((c * _BPC + bi) * 16, 16)] = vmax
            return jnp.maximum(vg, vmax)

        vglob = blk_body

    m_val = _lane_max(vglob)

    def red_body(b, bf):
        vec = bmax[pl.ds(b * 16, 16)]
        hitv = (vec == m_val) & (bf == _NBLK)
        return jnp.where(hitv, b, bf)

    bfirst = lax.fori_loop(
        0, _NBLK, red_body, jnp.full((16,), _NBLK, jnp.int32))
    b_star = (-_lane_max(-bfirst.astype(jnp.float32))).astype(jnp.int32)
    b_star = jnp.minimum(b_star, _NBLK - 1)

    gbase = base + b_star * _BLK
    cpa = pltpu.make_async_copy(
        in_hbm.at[pl.ds(gbase, _BLK)], in_a.at[pl.ds(0, _BLK)], sem_a)
    cpb = pltpu.make_async_copy(
        tg_hbm.at[pl.ds(gbase, _BLK)], tg_a.at[pl.ds(0, _BLK)], sem_a)
    cpa.start()
    cpb.start()
    cpa.wait()
    cpb.wait()

    lane = lax.iota(jnp.int32, 16)

    def rs_body(si, carry):
        bidx, b_a, b_b = carry
        a = in_a[pl.ds(si * 16, 16)]
        t = tg_a[pl.ds(si * 16, 16)]
        d = a - t
        d2 = d * d
        idxv = gbase + si * 16 + lane
        hit = (d2 == m_val) & (idxv < bidx)
        return (jnp.where(hit, idxv, bidx),
                jnp.where(hit, a, b_a),
                jnp.where(hit, t, b_b))

    bidx, b_a, b_b = lax.fori_loop(
        0, _SPB, rs_body,
        (jnp.full((16,), _BIGI, jnp.int32),
         jnp.zeros((16,), jnp.float32),
         jnp.zeros((16,), jnp.float32)))

    found = bidx < _BIGI
    vscr[...] = jnp.where(found, m_val, _NEG)
    iscr[...] = bidx
    ascr[...] = b_a
    bscr[...] = b_b
    pltpu.sync_copy(vscr, v_out.at[wid])
    pltpu.sync_copy(iscr, i_out.at[wid])
    pltpu.sync_copy(ascr, a_out.at[wid])
    pltpu.sync_copy(bscr, b_out.at[wid])


_phase1 = pl.kernel(
    _scan_body,
    out_type=[
        jax.ShapeDtypeStruct((_NW, 16), jnp.float32),
        jax.ShapeDtypeStruct((_NW, 16), jnp.int32),
        jax.ShapeDtypeStruct((_NW, 16), jnp.float32),
        jax.ShapeDtypeStruct((_NW, 16), jnp.float32),
    ],
    mesh=plsc.VectorSubcoreMesh(
        core_axis_name="c", subcore_axis_name="s",
        num_cores=_NC, num_subcores=_NS),
    scratch_types=[
        pltpu.VMEM((_CH,), jnp.float32),
        pltpu.VMEM((_CH,), jnp.float32),
        pltpu.VMEM((_CH,), jnp.float32),
        pltpu.VMEM((_CH,), jnp.float32),
        pltpu.VMEM((_NBLK * 16,), jnp.float32),
        pltpu.VMEM((16,), jnp.float32),
        pltpu.VMEM((16,), jnp.int32),
        pltpu.VMEM((16,), jnp.float32),
        pltpu.VMEM((16,), jnp.float32),
        pltpu.SemaphoreType.DMA,
        pltpu.SemaphoreType.DMA,
    ],
)


def _tc_body(in_ref, tg_ref, bm_ref, mo_ref, so_ref):
    i = pl.program_id(0)
    a = in_ref[...].reshape(_CB // 128, 128)
    t = tg_ref[...].reshape(_CB // 128, 128)
    d = a - t
    d2 = d * d
    for j in range(_SB):
        bm_ref[i * _SB + j] = jnp.max(d2[j * _BR:(j + 1) * _BR, :])

    @pl.when(i == _GT - 1)
    def _():
        def red(s, carry):
            m_cur, s_cur = carry
            m_s = bm_ref[s]
            take = m_s > m_cur
            return (jnp.where(take, m_s, m_cur), jnp.where(take, s, s_cur))

        m_tc, s_tc = lax.fori_loop(
            0, _NBM, red, (jnp.float32(-1.0), jnp.int32(0)))
        mo_ref[0] = m_tc
        so_ref[0] = s_tc


_tc_scan = pl.pallas_call(
    _tc_body,
    grid=(_GT,),
    in_specs=[
        pl.BlockSpec((_CB,), lambda i: (_B0 + i,)),
        pl.BlockSpec((_CB,), lambda i: (_B0 + i,)),
    ],
    out_shape=[
        jax.ShapeDtypeStruct((_NBM,), jnp.float32),
        jax.ShapeDtypeStruct((1,), jnp.float32),
        jax.ShapeDtypeStruct((1,), jnp.int32),
    ],
    out_specs=[
        pl.BlockSpec(memory_space=pltpu.SMEM),
        pl.BlockSpec(memory_space=pltpu.SMEM),
        pl.BlockSpec(memory_space=pltpu.SMEM),
    ],
)


def _merge_body(s_ref, in_ref, tg_ref, mt_ref, v_ref, i_ref, a_ref, b_ref,
                md_ref, p_ref, ac_ref):
    v = v_ref[...]
    idx = i_ref[...]
    m_sc = jnp.max(v)
    maskv = v == m_sc
    g_sc = jnp.min(jnp.where(maskv, idx, _BIGI))
    sel = maskv & (idx == g_sc)
    a_sc = jnp.max(jnp.where(sel, a_ref[...], _NEG))
    b_sc = jnp.max(jnp.where(sel, b_ref[...], _NEG))
    m_tc = mt_ref[0]
    a_blk = in_ref[...].reshape(_BR, 128)
    t_blk = tg_ref[...].reshape(_BR, 128)
    d = a_blk - t_blk
    d2 = d * d
    rows = lax.broadcasted_iota(jnp.int32, (_BR, 128), 0)
    cols = lax.broadcasted_iota(jnp.int32, (_BR, 128), 1)
    idxf = _NSC + s_ref[0] * _RB + rows * 128 + cols
    eq = d2 == m_tc
    g_tc = jnp.min(jnp.where(eq, idxf, _BIGI))
    selt = eq & (idxf == g_tc)
    a_tc = jnp.max(jnp.where(selt, a_blk, _NEG))
    b_tc = jnp.max(jnp.where(selt, t_blk, _NEG))
    take_tc = m_tc > m_sc
    md_ref[0] = jnp.sqrt(jnp.where(take_tc, m_tc, m_sc))
    p_ref[0] = jnp.where(take_tc, a_tc, a_sc)
    ac_ref[0] = jnp.where(take_tc, b_tc, b_sc)


_phase2 = pl.pallas_call(
    _merge_body,
    grid_spec=pltpu.PrefetchScalarGridSpec(
        num_scalar_prefetch=1,
        grid=(1,),
        in_specs=[
            pl.BlockSpec((_RB,), lambda i, s: (_NSC // _RB + s[0],)),
            pl.BlockSpec((_RB,), lambda i, s: (_NSC // _RB + s[0],)),
            pl.BlockSpec(memory_space=pltpu.SMEM),
            pl.BlockSpec(memory_space=pltpu.MemorySpace.VMEM),
            pl.BlockSpec(memory_space=pltpu.MemorySpace.VMEM),
            pl.BlockSpec(memory_space=pltpu.MemorySpace.VMEM),
            pl.BlockSpec(memory_space=pltpu.MemorySpace.VMEM),
        ],
        out_specs=[
            pl.BlockSpec(memory_space=pltpu.SMEM),
            pl.BlockSpec(memory_space=pltpu.SMEM),
            pl.BlockSpec(memory_space=pltpu.SMEM),
        ],
    ),
    out_shape=[
        jax.ShapeDtypeStruct((1,), jnp.float32),
        jax.ShapeDtypeStruct((1,), jnp.float32),
        jax.ShapeDtypeStruct((1,), jnp.float32),
    ],
)


def kernel(inputs, target):
    v, idx, a, b = _phase1(inputs, target)
    _, mt, st = _tc_scan(inputs, target)
    md, p, ac = _phase2(st, inputs, target, mt, v, idx, a, b)
    return (md[0], p[0], ac[0])


# speedup vs baseline: 1.3908x; 1.0710x over previous
import functools

import jax
import jax.numpy as jnp
from jax import lax
from jax.experimental import pallas as pl
from jax.experimental.pallas import tpu as pltpu
from jax.experimental.pallas import tpu_sc as plsc

_N = 4194304
_NC = 2
_NS = 16
_NW = _NC * _NS
_CH = 16384
_NCH = 2
_PW = _NCH * _CH
_NSC = _NW * _PW
_BLK = 1024
_SPB = _BLK // 16
_BPC = _CH // _BLK
_NBLK = _PW // _BLK

_CB = 524288
_RB = 65536
_SB = _CB // _RB
_BR = _RB // 128
_B0 = _NSC // _CB
_GT = (_N - _NSC) // _CB
_NBM = _GT * _SB

_NEG = -3.4e38
_BIGI = 2**30


def _lane_max(vec):
    idx = lax.iota(jnp.int32, 16)
    dnums = lax.GatherDimensionNumbers(
        offset_dims=(), collapsed_slice_dims=(0,), start_index_map=(0,))
    for sh in (8, 4, 2, 1):
        perm = jnp.bitwise_xor(idx, sh)
        shuf = lax.gather(vec, perm[:, None], dnums, slice_sizes=(1,),
                          unique_indices=True, indices_are_sorted=False,
                          mode=lax.GatherScatterMode.PROMISE_IN_BOUNDS)
        vec = jnp.maximum(vec, shuf)
    return vec[0]


def _scan_body(in_hbm, tg_hbm, v_out, i_out, a_out, b_out,
               in_a, in_b, tg_a, tg_b, bmax,
               vscr, iscr, ascr, bscr, sem_a, sem_b):
    cid = lax.axis_index("c")
    sid = lax.axis_index("s")
    wid = sid * _NC + cid
    base = wid * _PW

    in_bufs = (in_a, in_b)
    tg_bufs = (tg_a, tg_b)
    sems = (sem_a, sem_b)

    def fire(c, par):
        @pl.when(c < _NCH)
        def _():
            pltpu.make_async_copy(
                in_hbm.at[pl.ds(base + c * _CH, _CH)],
                in_bufs[par], sems[par]).start()
            pltpu.make_async_copy(
                tg_hbm.at[pl.ds(base + c * _CH, _CH)],
                tg_bufs[par], sems[par]).start()

    def wait(par):
        pltpu.make_async_copy(
            in_hbm.at[pl.ds(base, _CH)], in_bufs[par], sems[par]).wait()
        pltpu.make_async_copy(
            tg_hbm.at[pl.ds(base, _CH)], tg_bufs[par], sems[par]).wait()

    def scan_chunk(c, par, vg):
        ibuf = in_bufs[par]
        tbuf = tg_bufs[par]

        def blk_body(bi, vg2):
            def grp_body(g, accs):
                a0 = accs
                off0 = bi * _BLK + g * 128
                for s in range(8):
                    off = off0 + s * 16
                    a = ibuf[pl.ds(off, 16)]
                    t = tbuf[pl.ds(off, 16)]
                    d = a - t
                    a0 = tuple(
                        jnp.maximum(a0[k], d * d) if k == s % 4 else a0[k]
                        for k in range(4))
                return a0

            zero = jnp.zeros((16,), jnp.float32)
            accs = lax.fori_loop(0, _SPB // 8, grp_body,
                                 (zero, zero, zero, zero))
            vmax = jnp.maximum(jnp.maximum(accs[0], accs[1]),
                               jnp.maximum(accs[2], accs[3]))
            bmax[pl.ds((c * _BPC + bi) * 16, 16)] = vmax
            return jnp.maximum(vg2, vmax)

        return lax.fori_loop(0, _BPC, blk_body, vg)

    fire(0, 0)
    fire(1, 1)

    def pair_body(c2, vg):
        c0 = 2 * c2
        wait(0)
        vg = scan_chunk(c0, 0, vg)
        fire(c0 + 2, 0)
        wait(1)
        vg = scan_chunk(c0 + 1, 1, vg)
        fire(c0 + 3, 1)
        return vg

    vglob = lax.fori_loop(0, _NCH // 2, pair_body,
                          jnp.zeros((16,), jnp.float32))

    m_val = _lane_max(vglob)

    def red_body(b, bf):
        vec = bmax[pl.ds(b * 16, 16)]
        hitv = (vec == m_val) & (bf == _NBLK)
        return jnp.where(hitv, b, bf)

    bfirst = lax.fori_loop(
        0, _NBLK, red_body, jnp.full((16,), _NBLK, jnp.int32))
    b_star = (-_lane_max(-bfirst.astype(jnp.float32))).astype(jnp.int32)
    b_star = jnp.minimum(b_star, _NBLK - 1)

    gbase = base + b_star * _BLK
    cpa = pltpu.make_async_copy(
        in_hbm.at[pl.ds(gbase, _BLK)], in_a.at[pl.ds(0, _BLK)], sem_a)
    cpb = pltpu.make_async_copy(
        tg_hbm.at[pl.ds(gbase, _BLK)], tg_a.at[pl.ds(0, _BLK)], sem_a)
    cpa.start()
    cpb.start()
    cpa.wait()
    cpb.wait()

    lane = lax.iota(jnp.int32, 16)

    def rs_body(si, carry):
        bidx, b_a, b_b = carry
        a = in_a[pl.ds(si * 16, 16)]
        t = tg_a[pl.ds(si * 16, 16)]
        d = a - t
        d2 = d * d
        idxv = gbase + si * 16 + lane
        hit = (d2 == m_val) & (idxv < bidx)
        return (jnp.where(hit, idxv, bidx),
                jnp.where(hit, a, b_a),
                jnp.where(hit, t, b_b))

    bidx, b_a, b_b = lax.fori_loop(
        0, _SPB, rs_body,
        (jnp.full((16,), _BIGI, jnp.int32),
         jnp.zeros((16,), jnp.float32),
         jnp.zeros((16,), jnp.float32)))

    found = bidx < _BIGI
    vscr[...] = jnp.where(found, m_val, _NEG)
    iscr[...] = bidx
    ascr[...] = b_a
    bscr[...] = b_b
    pltpu.sync_copy(vscr, v_out.at[wid])
    pltpu.sync_copy(iscr, i_out.at[wid])
    pltpu.sync_copy(ascr, a_out.at[wid])
    pltpu.sync_copy(bscr, b_out.at[wid])


_phase1 = pl.kernel(
    _scan_body,
    out_type=[
        jax.ShapeDtypeStruct((_NW, 16), jnp.float32),
        jax.ShapeDtypeStruct((_NW, 16), jnp.int32),
        jax.ShapeDtypeStruct((_NW, 16), jnp.float32),
        jax.ShapeDtypeStruct((_NW, 16), jnp.float32),
    ],
    mesh=plsc.VectorSubcoreMesh(
        core_axis_name="c", subcore_axis_name="s",
        num_cores=_NC, num_subcores=_NS),
    scratch_types=[
        pltpu.VMEM((_CH,), jnp.float32),
        pltpu.VMEM((_CH,), jnp.float32),
        pltpu.VMEM((_CH,), jnp.float32),
        pltpu.VMEM((_CH,), jnp.float32),
        pltpu.VMEM((_NBLK * 16,), jnp.float32),
        pltpu.VMEM((16,), jnp.float32),
        pltpu.VMEM((16,), jnp.int32),
        pltpu.VMEM((16,), jnp.float32),
        pltpu.VMEM((16,), jnp.float32),
        pltpu.SemaphoreType.DMA,
        pltpu.SemaphoreType.DMA,
    ],
)


def _tc_body(in_ref, tg_ref, bm_ref, mo_ref, so_ref):
    i = pl.program_id(0)
    a = in_ref[...].reshape(_CB // 128, 128)
    t = tg_ref[...].reshape(_CB // 128, 128)
    d = a - t
    d2 = d * d
    for j in range(_SB):
        bm_ref[i * _SB + j] = jnp.max(d2[j * _BR:(j + 1) * _BR, :])

    @pl.when(i == _GT - 1)
    def _():
        def red(s, carry):
            m_cur, s_cur = carry
            m_s = bm_ref[s]
            take = m_s > m_cur
            return (jnp.where(take, m_s, m_cur), jnp.where(take, s, s_cur))

        m_tc, s_tc = lax.fori_loop(
            0, _NBM, red, (jnp.float32(-1.0), jnp.int32(0)))
        mo_ref[0] = m_tc
        so_ref[0] = s_tc


_tc_scan = pl.pallas_call(
    _tc_body,
    grid=(_GT,),
    in_specs=[
        pl.BlockSpec((_CB,), lambda i: (_B0 + i,)),
        pl.BlockSpec((_CB,), lambda i: (_B0 + i,)),
    ],
    out_shape=[
        jax.ShapeDtypeStruct((_NBM,), jnp.float32),
        jax.ShapeDtypeStruct((1,), jnp.float32),
        jax.ShapeDtypeStruct((1,), jnp.int32),
    ],
    out_specs=[
        pl.BlockSpec(memory_space=pltpu.SMEM),
        pl.BlockSpec(memory_space=pltpu.SMEM),
        pl.BlockSpec(memory_space=pltpu.SMEM),
    ],
)


def _merge_body(s_ref, in_ref, tg_ref, mt_ref, v_ref, i_ref, a_ref, b_ref,
                md_ref, p_ref, ac_ref):
    v = v_ref[...]
    idx = i_ref[...]
    m_sc = jnp.max(v)
    maskv = v == m_sc
    g_sc = jnp.min(jnp.where(maskv, idx, _BIGI))
    sel = maskv & (idx == g_sc)
    a_sc = jnp.max(jnp.where(sel, a_ref[...], _NEG))
    b_sc = jnp.max(jnp.where(sel, b_ref[...], _NEG))
    m_tc = mt_ref[0]
    a_blk = in_ref[...].reshape(_BR, 128)
    t_blk = tg_ref[...].reshape(_BR, 128)
    d = a_blk - t_blk
    d2 = d * d
    rows = lax.broadcasted_iota(jnp.int32, (_BR, 128), 0)
    cols = lax.broadcasted_iota(jnp.int32, (_BR, 128), 1)
    idxf = _NSC + s_ref[0] * _RB + rows * 128 + cols
    eq = d2 == m_tc
    g_tc = jnp.min(jnp.where(eq, idxf, _BIGI))
    selt = eq & (idxf == g_tc)
    a_tc = jnp.max(jnp.where(selt, a_blk, _NEG))
    b_tc = jnp.max(jnp.where(selt, t_blk, _NEG))
    take_tc = m_tc > m_sc
    md_ref[0] = jnp.sqrt(jnp.where(take_tc, m_tc, m_sc))
    p_ref[0] = jnp.where(take_tc, a_tc, a_sc)
    ac_ref[0] = jnp.where(take_tc, b_tc, b_sc)


_phase2 = pl.pallas_call(
    _merge_body,
    grid_spec=pltpu.PrefetchScalarGridSpec(
        num_scalar_prefetch=1,
        grid=(1,),
        in_specs=[
            pl.BlockSpec((_RB,), lambda i, s: (_NSC // _RB + s[0],)),
            pl.BlockSpec((_RB,), lambda i, s: (_NSC // _RB + s[0],)),
            pl.BlockSpec(memory_space=pltpu.SMEM),
            pl.BlockSpec(memory_space=pltpu.MemorySpace.VMEM),
            pl.BlockSpec(memory_space=pltpu.MemorySpace.VMEM),
            pl.BlockSpec(memory_space=pltpu.MemorySpace.VMEM),
            pl.BlockSpec(memory_space=pltpu.MemorySpace.VMEM),
        ],
        out_specs=[
            pl.BlockSpec(memory_space=pltpu.SMEM),
            pl.BlockSpec(memory_space=pltpu.SMEM),
            pl.BlockSpec(memory_space=pltpu.SMEM),
        ],
    ),
    out_shape=[
        jax.ShapeDtypeStruct((1,), jnp.float32),
        jax.ShapeDtypeStruct((1,), jnp.float32),
        jax.ShapeDtypeStruct((1,), jnp.float32),
    ],
)


def kernel(inputs, target):
    v, idx, a, b = _phase1(inputs, target)
    _, mt, st = _tc_scan(inputs, target)
    md, p, ac = _phase2(st, inputs, target, mt, v, idx, a, b)
    return (md[0], p[0], ac[0])


# speedup vs baseline: 1.4161x; 1.0182x over previous
import functools

import jax
import jax.numpy as jnp
from jax import lax
from jax.experimental import pallas as pl
from jax.experimental.pallas import tpu as pltpu
from jax.experimental.pallas import tpu_sc as plsc

_N = 4194304
_NC = 2
_NS = 16
_NW = _NC * _NS
_CH = 16384
_NCH = 2
_PW = _NCH * _CH
_NSC = _NW * _PW
_CB = 524288
_RB = 16384
_SB = _CB // _RB
_BR = _RB // 128
_B0 = _NSC // _CB
_GT = (_N - _NSC) // _CB
_NBM = _GT * _SB

_NEG = -3.4e38
_BIGI = 2**30


def _scan_body(in_hbm, tg_hbm, cand_out, ci_out,
               in_a, in_b, tg_a, tg_b, oscr, iscr, sem_a, sem_b):
    cid = lax.axis_index("c")
    sid = lax.axis_index("s")
    wid = sid * _NC + cid
    base = wid * _PW

    in_bufs = (in_a, in_b)
    tg_bufs = (tg_a, tg_b)
    sems = (sem_a, sem_b)
    lane = lax.iota(jnp.int32, 16)

    def fire(c, par):
        @pl.when(c < _NCH)
        def _():
            pltpu.make_async_copy(
                in_hbm.at[pl.ds(base + c * _CH, _CH)],
                in_bufs[par], sems[par]).start()
            pltpu.make_async_copy(
                tg_hbm.at[pl.ds(base + c * _CH, _CH)],
                tg_bufs[par], sems[par]).start()

    def wait(par):
        pltpu.make_async_copy(
            in_hbm.at[pl.ds(base, _CH)], in_bufs[par], sems[par]).wait()
        pltpu.make_async_copy(
            tg_hbm.at[pl.ds(base, _CH)], tg_bufs[par], sems[par]).wait()

    def scan_chunk(c, par, carry):
        ibuf = in_bufs[par]
        tbuf = tg_bufs[par]

        def grp_body(g, cr):
            vmax, vidx, va, vb = cr
            off0 = g * 128
            gidx0 = base + c * _CH + off0
            for s in range(8):
                off = off0 + s * 16
                a = ibuf[pl.ds(off, 16)]
                t = tbuf[pl.ds(off, 16)]
                d = a - t
                d2 = d * d
                hit = d2 > vmax
                idxv = gidx0 + s * 16 + lane
                vmax = jnp.where(hit, d2, vmax)
                vidx = jnp.where(hit, idxv, vidx)
                va = jnp.where(hit, a, va)
                vb = jnp.where(hit, t, vb)
            return (vmax, vidx, va, vb)

        return lax.fori_loop(0, _CH // 128, grp_body, carry)

    fire(0, 0)
    fire(1, 1)

    init = (jnp.full((16,), -1.0, jnp.float32),
            jnp.full((16,), _BIGI, jnp.int32),
            jnp.zeros((16,), jnp.float32),
            jnp.zeros((16,), jnp.float32))

    def pair_body(c2, cr):
        c0 = 2 * c2
        wait(0)
        cr = scan_chunk(c0, 0, cr)
        fire(c0 + 2, 0)
        wait(1)
        cr = scan_chunk(c0 + 1, 1, cr)
        fire(c0 + 3, 1)
        return cr

    vmax, vidx, va, vb = lax.fori_loop(0, _NCH // 2, pair_body, init)

    oscr[pl.ds(0, 16)] = vmax
    oscr[pl.ds(16, 16)] = va
    oscr[pl.ds(32, 16)] = vb
    iscr[...] = vidx
    pltpu.sync_copy(oscr, cand_out.at[wid])
    pltpu.sync_copy(iscr, ci_out.at[wid])


_phase1 = pl.kernel(
    _scan_body,
    out_type=[
        jax.ShapeDtypeStruct((_NW, 48), jnp.float32),
        jax.ShapeDtypeStruct((_NW, 16), jnp.int32),
    ],
    mesh=plsc.VectorSubcoreMesh(
        core_axis_name="c", subcore_axis_name="s",
        num_cores=_NC, num_subcores=_NS),
    scratch_types=[
        pltpu.VMEM((_CH,), jnp.float32),
        pltpu.VMEM((_CH,), jnp.float32),
        pltpu.VMEM((_CH,), jnp.float32),
        pltpu.VMEM((_CH,), jnp.float32),
        pltpu.VMEM((48,), jnp.float32),
        pltpu.VMEM((16,), jnp.int32),
        pltpu.SemaphoreType.DMA,
        pltpu.SemaphoreType.DMA,
    ],
)


def _tc_body(in_ref, tg_ref, bm_ref, mo_ref, so_ref):
    i = pl.program_id(0)
    a = in_ref[...].reshape(_CB // 128, 128)
    t = tg_ref[...].reshape(_CB // 128, 128)
    d = a - t
    d2 = d * d
    for j in range(_SB):
        bm_ref[i * _SB + j] = jnp.max(d2[j * _BR:(j + 1) * _BR, :])

    @pl.when(i == _GT - 1)
    def _():
        def red(s, carry):
            m_cur, s_cur = carry
            m_s = bm_ref[s]
            take = m_s > m_cur
            return (jnp.where(take, m_s, m_cur), jnp.where(take, s, s_cur))

        m_tc, s_tc = lax.fori_loop(
            0, _NBM, red, (jnp.float32(-1.0), jnp.int32(0)))
        mo_ref[0] = m_tc
        so_ref[0] = s_tc


_tc_scan = pl.pallas_call(
    _tc_body,
    grid=(_GT,),
    in_specs=[
        pl.BlockSpec((_CB,), lambda i: (_B0 + i,)),
        pl.BlockSpec((_CB,), lambda i: (_B0 + i,)),
    ],
    out_shape=[
        jax.ShapeDtypeStruct((_NBM,), jnp.float32),
        jax.ShapeDtypeStruct((1,), jnp.float32),
        jax.ShapeDtypeStruct((1,), jnp.int32),
    ],
    out_specs=[
        pl.BlockSpec(memory_space=pltpu.SMEM),
        pl.BlockSpec(memory_space=pltpu.SMEM),
        pl.BlockSpec(memory_space=pltpu.SMEM),
    ],
)


def _merge_body(s_ref, in_ref, tg_ref, mt_ref, c_ref, ci_ref,
                md_ref, p_ref, ac_ref):
    cand = c_ref[...]
    v = cand[:, 0:16]
    idx = ci_ref[...]
    m_sc = jnp.max(v)
    maskv = v == m_sc
    g_sc = jnp.min(jnp.where(maskv, idx, _BIGI))
    sel = maskv & (idx == g_sc)
    a_sc = jnp.max(jnp.where(sel, cand[:, 16:32], _NEG))
    b_sc = jnp.max(jnp.where(sel, cand[:, 32:48], _NEG))
    m_tc = mt_ref[0]
    a_blk = in_ref[...].reshape(_BR, 128)
    t_blk = tg_ref[...].reshape(_BR, 128)
    d = a_blk - t_blk
    d2 = d * d
    rows = lax.broadcasted_iota(jnp.int32, (_BR, 128), 0)
    cols = lax.broadcasted_iota(jnp.int32, (_BR, 128), 1)
    idxf = _NSC + s_ref[0] * _RB + rows * 128 + cols
    eq = d2 == m_tc
    g_tc = jnp.min(jnp.where(eq, idxf, _BIGI))
    selt = eq & (idxf == g_tc)
    a_tc = jnp.max(jnp.where(selt, a_blk, _NEG))
    b_tc = jnp.max(jnp.where(selt, t_blk, _NEG))
    take_tc = m_tc > m_sc
    md_ref[0] = jnp.sqrt(jnp.where(take_tc, m_tc, m_sc))
    p_ref[0] = jnp.where(take_tc, a_tc, a_sc)
    ac_ref[0] = jnp.where(take_tc, b_tc, b_sc)


_phase2 = pl.pallas_call(
    _merge_body,
    grid_spec=pltpu.PrefetchScalarGridSpec(
        num_scalar_prefetch=1,
        grid=(1,),
        in_specs=[
            pl.BlockSpec((_RB,), lambda i, s: (_NSC // _RB + s[0],)),
            pl.BlockSpec((_RB,), lambda i, s: (_NSC // _RB + s[0],)),
            pl.BlockSpec(memory_space=pltpu.SMEM),
            pl.BlockSpec(memory_space=pltpu.MemorySpace.VMEM),
            pl.BlockSpec(memory_space=pltpu.MemorySpace.VMEM),
        ],
        out_specs=[
            pl.BlockSpec(memory_space=pltpu.SMEM),
            pl.BlockSpec(memory_space=pltpu.SMEM),
            pl.BlockSpec(memory_space=pltpu.SMEM),
        ],
    ),
    out_shape=[
        jax.ShapeDtypeStruct((1,), jnp.float32),
        jax.ShapeDtypeStruct((1,), jnp.float32),
        jax.ShapeDtypeStruct((1,), jnp.float32),
    ],
)


def kernel(inputs, target):
    cand, cidx = _phase1(inputs, target)
    _, mt, st = _tc_scan(inputs, target)
    md, p, ac = _phase2(st, inputs, target, mt, cand, cidx)
    return (md[0], p[0], ac[0])


# speedup vs baseline: 1.4235x; 1.0052x over previous
import functools

import jax
import jax.numpy as jnp
from jax import lax
from jax.experimental import pallas as pl
from jax.experimental.pallas import tpu as pltpu
from jax.experimental.pallas import tpu_sc as plsc

_N = 4194304
_NC = 2
_NS = 16
_NW = _NC * _NS
_CH = 16384
_NCH = 2
_PW = _NCH * _CH
_NSC = _NW * _PW
_CB = 1048576
_RB = 16384
_SB = _CB // _RB
_BR = _RB // 128
_B0 = _NSC // _CB
_GT = (_N - _NSC) // _CB
_NBM = _GT * _SB

_NEG = -3.4e38
_BIGI = 2**30


def _scan_body(in_hbm, tg_hbm, cand_out, ci_out,
               in_a, in_b, tg_a, tg_b, oscr, iscr, sem_a, sem_b):
    cid = lax.axis_index("c")
    sid = lax.axis_index("s")
    wid = sid * _NC + cid
    base = wid * _PW

    in_bufs = (in_a, in_b)
    tg_bufs = (tg_a, tg_b)
    sems = (sem_a, sem_b)
    lane = lax.iota(jnp.int32, 16)

    def fire(c, par):
        @pl.when(c < _NCH)
        def _():
            pltpu.make_async_copy(
                in_hbm.at[pl.ds(base + c * _CH, _CH)],
                in_bufs[par], sems[par]).start()
            pltpu.make_async_copy(
                tg_hbm.at[pl.ds(base + c * _CH, _CH)],
                tg_bufs[par], sems[par]).start()

    def wait(par):
        pltpu.make_async_copy(
            in_hbm.at[pl.ds(base, _CH)], in_bufs[par], sems[par]).wait()
        pltpu.make_async_copy(
            tg_hbm.at[pl.ds(base, _CH)], tg_bufs[par], sems[par]).wait()

    def scan_chunk(c, par, carry):
        ibuf = in_bufs[par]
        tbuf = tg_bufs[par]

        def grp_body(g, cr):
            vmax, vidx, va, vb = cr
            off0 = g * 128
            gidx0 = base + c * _CH + off0
            for s in range(8):
                off = off0 + s * 16
                a = ibuf[pl.ds(off, 16)]
                t = tbuf[pl.ds(off, 16)]
                d = a - t
                d2 = d * d
                hit = d2 > vmax
                idxv = gidx0 + s * 16 + lane
                vmax = jnp.where(hit, d2, vmax)
                vidx = jnp.where(hit, idxv, vidx)
                va = jnp.where(hit, a, va)
                vb = jnp.where(hit, t, vb)
            return (vmax, vidx, va, vb)

        return lax.fori_loop(0, _CH // 128, grp_body, carry)

    fire(0, 0)
    fire(1, 1)

    init = (jnp.full((16,), -1.0, jnp.float32),
            jnp.full((16,), _BIGI, jnp.int32),
            jnp.zeros((16,), jnp.float32),
            jnp.zeros((16,), jnp.float32))

    def pair_body(c2, cr):
        c0 = 2 * c2
        wait(0)
        cr = scan_chunk(c0, 0, cr)
        fire(c0 + 2, 0)
        wait(1)
        cr = scan_chunk(c0 + 1, 1, cr)
        fire(c0 + 3, 1)
        return cr

    vmax, vidx, va, vb = lax.fori_loop(0, _NCH // 2, pair_body, init)

    oscr[pl.ds(0, 16)] = vmax
    oscr[pl.ds(16, 16)] = va
    oscr[pl.ds(32, 16)] = vb
    iscr[...] = vidx
    pltpu.sync_copy(oscr, cand_out.at[wid])
    pltpu.sync_copy(iscr, ci_out.at[wid])


_phase1 = pl.kernel(
    _scan_body,
    out_type=[
        jax.ShapeDtypeStruct((_NW, 48), jnp.float32),
        jax.ShapeDtypeStruct((_NW, 16), jnp.int32),
    ],
    mesh=plsc.VectorSubcoreMesh(
        core_axis_name="c", subcore_axis_name="s",
        num_cores=_NC, num_subcores=_NS),
    scratch_types=[
        pltpu.VMEM((_CH,), jnp.float32),
        pltpu.VMEM((_CH,), jnp.float32),
        pltpu.VMEM((_CH,), jnp.float32),
        pltpu.VMEM((_CH,), jnp.float32),
        pltpu.VMEM((48,), jnp.float32),
        pltpu.VMEM((16,), jnp.int32),
        pltpu.SemaphoreType.DMA,
        pltpu.SemaphoreType.DMA,
    ],
)


def _tc_body(in_ref, tg_ref, bm_ref, mo_ref, so_ref):
    i = pl.program_id(0)
    a = in_ref[...].reshape(_CB // 128, 128)
    t = tg_ref[...].reshape(_CB // 128, 128)
    d = a - t
    d2 = d * d
    for j in range(_SB):
        bm_ref[i * _SB + j] = jnp.max(d2[j * _BR:(j + 1) * _BR, :])

    @pl.when(i == _GT - 1)
    def _():
        def red(s, carry):
            m_cur, s_cur = carry
            m_s = bm_ref[s]
            take = m_s > m_cur
            return (jnp.where(take, m_s, m_cur), jnp.where(take, s, s_cur))

        m_tc, s_tc = lax.fori_loop(
            0, _NBM, red, (jnp.float32(-1.0), jnp.int32(0)))
        mo_ref[0] = m_tc
        so_ref[0] = s_tc


_tc_scan = pl.pallas_call(
    _tc_body,
    grid=(_GT,),
    in_specs=[
        pl.BlockSpec((_CB,), lambda i: (_B0 + i,)),
        pl.BlockSpec((_CB,), lambda i: (_B0 + i,)),
    ],
    out_shape=[
        jax.ShapeDtypeStruct((_NBM,), jnp.float32),
        jax.ShapeDtypeStruct((1,), jnp.float32),
        jax.ShapeDtypeStruct((1,), jnp.int32),
    ],
    out_specs=[
        pl.BlockSpec(memory_space=pltpu.SMEM),
        pl.BlockSpec(memory_space=pltpu.SMEM),
        pl.BlockSpec(memory_space=pltpu.SMEM),
    ],
)


def _merge_body(s_ref, in_ref, tg_ref, mt_ref, c_ref, ci_ref,
                md_ref, p_ref, ac_ref):
    cand = c_ref[...]
    v = cand[:, 0:16]
    idx = ci_ref[...]
    m_sc = jnp.max(v)
    maskv = v == m_sc
    g_sc = jnp.min(jnp.where(maskv, idx, _BIGI))
    sel = maskv & (idx == g_sc)
    a_sc = jnp.max(jnp.where(sel, cand[:, 16:32], _NEG))
    b_sc = jnp.max(jnp.where(sel, cand[:, 32:48], _NEG))
    m_tc = mt_ref[0]
    a_blk = in_ref[...].reshape(_BR, 128)
    t_blk = tg_ref[...].reshape(_BR, 128)
    d = a_blk - t_blk
    d2 = d * d
    rows = lax.broadcasted_iota(jnp.int32, (_BR, 128), 0)
    cols = lax.broadcasted_iota(jnp.int32, (_BR, 128), 1)
    idxf = _NSC + s_ref[0] * _RB + rows * 128 + cols
    eq = d2 == m_tc
    g_tc = jnp.min(jnp.where(eq, idxf, _BIGI))
    selt = eq & (idxf == g_tc)
    a_tc = jnp.max(jnp.where(selt, a_blk, _NEG))
    b_tc = jnp.max(jnp.where(selt, t_blk, _NEG))
    take_tc = m_tc > m_sc
    md_ref[0] = jnp.sqrt(jnp.where(take_tc, m_tc, m_sc))
    p_ref[0] = jnp.where(take_tc, a_tc, a_sc)
    ac_ref[0] = jnp.where(take_tc, b_tc, b_sc)


_phase2 = pl.pallas_call(
    _merge_body,
    grid_spec=pltpu.PrefetchScalarGridSpec(
        num_scalar_prefetch=1,
        grid=(1,),
        in_specs=[
            pl.BlockSpec((_RB,), lambda i, s: (_NSC // _RB + s[0],)),
            pl.BlockSpec((_RB,), lambda i, s: (_NSC // _RB + s[0],)),
            pl.BlockSpec(memory_space=pltpu.SMEM),
            pl.BlockSpec(memory_space=pltpu.MemorySpace.VMEM),
            pl.BlockSpec(memory_space=pltpu.MemorySpace.VMEM),
        ],
        out_specs=[
            pl.BlockSpec(memory_space=pltpu.SMEM),
            pl.BlockSpec(memory_space=pltpu.SMEM),
            pl.BlockSpec(memory_space=pltpu.SMEM),
        ],
    ),
    out_shape=[
        jax.ShapeDtypeStruct((1,), jnp.float32),
        jax.ShapeDtypeStruct((1,), jnp.float32),
        jax.ShapeDtypeStruct((1,), jnp.float32),
    ],
)


def kernel(inputs, target):
    cand, cidx = _phase1(inputs, target)
    _, mt, st = _tc_scan(inputs, target)
    md, p, ac = _phase2(st, inputs, target, mt, cand, cidx)
    return (md[0], p[0], ac[0])
